# baseline (speedup 1.0000x reference)
import functools

import jax
import jax.numpy as jnp
from jax import lax
from jax.experimental import pallas as pl
from jax.experimental.pallas import tpu as pltpu
from jax.experimental.pallas import tpu_sc as plsc

ROWS = 16384
COLS = 512
K = 128
NUM_CORES = 2
NUM_SUBCORES = 16
NW = NUM_CORES * NUM_SUBCORES

SC_ROWS = 10240
TC_ROWS = ROWS - SC_ROWS
RPW = SC_ROWS // NW
CHUNK = 64
NCHUNK = RPW // CHUNK
DEPTH = 2
LANES = 16
NGRP = K // LANES

TC_BLK = 512
SC_BLKS = SC_ROWS // TC_BLK


def _sc_feature_select(values, indices):
    mesh = plsc.VectorSubcoreMesh(core_axis_name="c", subcore_axis_name="s")

    @functools.partial(
        pl.kernel,
        out_type=jax.ShapeDtypeStruct((SC_ROWS, K), jnp.float32),
        mesh=mesh,
        compiler_params=pltpu.CompilerParams(
            use_tc_tiling_on_sc=True, needs_layout_passes=False
        ),
        scratch_types=[
            pltpu.VMEM((K,), jnp.int32),
            pltpu.VMEM((DEPTH, CHUNK, COLS), jnp.float32),
            pltpu.VMEM((DEPTH, CHUNK, K), jnp.float32),
        ]
        + [pltpu.SemaphoreType.DMA] * (2 * DEPTH),
    )
    def body(values_hbm, idx_hbm, out_hbm, idx_v, in_v, out_v, *sems):
        sems_in = sems[:DEPTH]
        sems_out = sems[DEPTH:]
        wid = lax.axis_index("s") * NUM_CORES + lax.axis_index("c")
        row0 = wid * RPW

        pltpu.sync_copy(idx_hbm, idx_v)
        idx_regs = [idx_v[pl.ds(g * LANES, LANES)] for g in range(NGRP)]

        def start_in(ck, sl):
            pltpu.async_copy(
                values_hbm.at[pl.ds(row0 + ck * CHUNK, CHUNK), :],
                in_v.at[sl], sems_in[sl])

        def start_out(ck, sl):
            pltpu.async_copy(
                out_v.at[sl],
                out_hbm.at[pl.ds(row0 + ck * CHUNK, CHUNK), :],
                sems_out[sl])

        def wait_in(sl):
            pltpu.make_async_copy(
                values_hbm.at[pl.ds(row0, CHUNK), :], in_v.at[sl],
                sems_in[sl]).wait()

        def wait_out(sl):
            pltpu.make_async_copy(
                out_v.at[sl], out_hbm.at[pl.ds(row0, CHUNK), :],
                sems_out[sl]).wait()

        for sl in range(DEPTH):
            start_in(sl, sl)

        for ck in range(NCHUNK):
            sl = ck % DEPTH
            wait_in(sl)
            if ck >= DEPTH:
                wait_out(sl)

            in_blk = in_v.at[sl]
            out_blk = out_v.at[sl]

            @plsc.parallel_loop(0, CHUNK, step=1, unroll=4)
            def row_body(r):
                rvec = jnp.full((LANES,), r, jnp.int32)
                for g in range(NGRP):
                    v = plsc.load_gather(in_blk, [rvec, idx_regs[g]])
                    out_blk[r, pl.ds(g * LANES, LANES)] = v

            start_out(ck, sl)
            if ck + DEPTH < NCHUNK:
                start_in(ck + DEPTH, sl)

        for sl in range(min(DEPTH, NCHUNK)):
            wait_out(sl)

    return body(values, indices)


def _tc_body(v_ref, idx_ref, o_ref):
    col = lax.broadcasted_iota(jnp.int32, (COLS, K), 0)
    onehot = (col == idx_ref[...][None, :]).astype(jnp.float32)
    o_ref[...] = jnp.dot(
        v_ref[...], onehot,
        preferred_element_type=jnp.float32,
        precision=lax.Precision.HIGHEST,
    )


def _tc_feature_select(values, indices):
    return pl.pallas_call(
        _tc_body,
        grid=(TC_ROWS // TC_BLK,),
        in_specs=[
            pl.BlockSpec((TC_BLK, COLS), lambda i: (SC_BLKS + i, 0)),
            pl.BlockSpec((K,), lambda i: (0,)),
        ],
        out_specs=pl.BlockSpec((TC_BLK, K), lambda i: (i, 0)),
        out_shape=jax.ShapeDtypeStruct((TC_ROWS, K), jnp.float32),
    )(values, indices)


def kernel(values, indices):
    sc_out = _sc_feature_select(values, indices)
    tc_out = _tc_feature_select(values, indices)
    return jnp.concatenate([sc_out, tc_out], axis=0)


# speedup vs baseline: 1.0415x; 1.0415x over previous
import functools

import jax
import jax.numpy as jnp
from jax import lax
from jax.experimental import pallas as pl
from jax.experimental.pallas import tpu as pltpu
from jax.experimental.pallas import tpu_sc as plsc

ROWS = 16384
COLS = 512
K = 128
NUM_CORES = 2
NUM_SUBCORES = 16
NW = NUM_CORES * NUM_SUBCORES
RPW = ROWS // NW
CHUNK = 64
NCHUNK = RPW // CHUNK
DEPTH = 2
LANES = 16
NGRP = K // LANES


def _sc_feature_select(values, indices):
    mesh = plsc.VectorSubcoreMesh(
        core_axis_name="c", subcore_axis_name="s", num_cores=NUM_CORES
    )

    @functools.partial(
        pl.kernel,
        out_type=jax.ShapeDtypeStruct((ROWS, K), jnp.float32),
        mesh=mesh,
        compiler_params=pltpu.CompilerParams(
            use_tc_tiling_on_sc=True, needs_layout_passes=False
        ),
        scratch_types=[
            pltpu.VMEM((K,), jnp.int32),
            pltpu.VMEM((DEPTH, CHUNK, COLS), jnp.float32),
            pltpu.VMEM((DEPTH, CHUNK, K), jnp.float32),
        ]
        + [pltpu.SemaphoreType.DMA] * (2 * DEPTH),
    )
    def body(values_hbm, idx_hbm, out_hbm, idx_v, in_v, out_v, *sems):
        sems_in = sems[:DEPTH]
        sems_out = sems[DEPTH:]
        wid = lax.axis_index("s") * NUM_CORES + lax.axis_index("c")
        row0 = wid * RPW

        pltpu.sync_copy(idx_hbm, idx_v)
        idx_regs = [idx_v[pl.ds(g * LANES, LANES)] for g in range(NGRP)]

        def start_in(ck, sl):
            pltpu.async_copy(
                values_hbm.at[pl.ds(row0 + ck * CHUNK, CHUNK), :],
                in_v.at[sl], sems_in[sl])

        def start_out(ck, sl):
            pltpu.async_copy(
                out_v.at[sl],
                out_hbm.at[pl.ds(row0 + ck * CHUNK, CHUNK), :],
                sems_out[sl])

        def wait_in(sl):
            pltpu.make_async_copy(
                values_hbm.at[pl.ds(row0, CHUNK), :], in_v.at[sl],
                sems_in[sl]).wait()

        def wait_out(sl):
            pltpu.make_async_copy(
                out_v.at[sl], out_hbm.at[pl.ds(row0, CHUNK), :],
                sems_out[sl]).wait()

        for sl in range(DEPTH):
            start_in(sl, sl)

        for ck in range(NCHUNK):
            sl = ck % DEPTH
            wait_in(sl)
            if ck >= DEPTH:
                wait_out(sl)

            in_blk = in_v.at[sl]
            out_blk = out_v.at[sl]

            @plsc.parallel_loop(0, CHUNK, step=1, unroll=4)
            def row_body(r):
                rvec = jnp.full((LANES,), r, jnp.int32)
                for g in range(NGRP):
                    v = plsc.load_gather(in_blk, [rvec, idx_regs[g]])
                    out_blk[r, pl.ds(g * LANES, LANES)] = v

            start_out(ck, sl)
            if ck + DEPTH < NCHUNK:
                start_in(ck + DEPTH, sl)

        for sl in range(min(DEPTH, NCHUNK)):
            wait_out(sl)

    return body(values, indices)


def kernel(values, indices):
    return _sc_feature_select(values, indices)


# speedup vs baseline: 1.0992x; 1.0555x over previous
import functools

import jax
import jax.numpy as jnp
from jax import lax
from jax.experimental import pallas as pl
from jax.experimental.pallas import tpu as pltpu
from jax.experimental.pallas import tpu_sc as plsc

ROWS = 16384
COLS = 512
K = 128
NUM_CORES = 2
NUM_SUBCORES = 16
NW = NUM_CORES * NUM_SUBCORES
RPW = ROWS // NW
CHUNK = 64
NCHUNK = RPW // CHUNK
DEPTH = 2
LANES = 16
NGRP = K // LANES


def _sc_feature_select(values, indices):
    mesh = plsc.VectorSubcoreMesh(
        core_axis_name="c", subcore_axis_name="s", num_cores=NUM_CORES
    )

    @functools.partial(
        pl.kernel,
        out_type=jax.ShapeDtypeStruct((ROWS, K), jnp.float32),
        mesh=mesh,
        compiler_params=pltpu.CompilerParams(
            use_tc_tiling_on_sc=True, needs_layout_passes=False
        ),
        scratch_types=[
            pltpu.VMEM((K,), jnp.int32),
            pltpu.VMEM((DEPTH, CHUNK, COLS), jnp.float32),
            pltpu.VMEM((DEPTH, CHUNK, K), jnp.float32),
        ]
        + [pltpu.SemaphoreType.DMA] * (2 * DEPTH),
    )
    def body(values_hbm, idx_hbm, out_hbm, idx_v, in_v, out_v, *sems):
        sems_in = sems[:DEPTH]
        sems_out = sems[DEPTH:]
        wid = lax.axis_index("s") * NUM_CORES + lax.axis_index("c")
        row0 = wid * RPW

        pltpu.sync_copy(idx_hbm, idx_v)
        idx_regs = [idx_v[pl.ds(g * LANES, LANES)] for g in range(NGRP)]

        def start_in(ck, sl):
            pltpu.async_copy(
                values_hbm.at[pl.ds(row0 + ck * CHUNK, CHUNK), :],
                in_v.at[sl], sems_in[sl])

        def start_out(ck, sl):
            pltpu.async_copy(
                out_v.at[sl],
                out_hbm.at[pl.ds(row0 + ck * CHUNK, CHUNK), :],
                sems_out[sl])

        def wait_in(sl):
            pltpu.make_async_copy(
                values_hbm.at[pl.ds(row0, CHUNK), :], in_v.at[sl],
                sems_in[sl]).wait()

        def wait_out(sl):
            pltpu.make_async_copy(
                out_v.at[sl], out_hbm.at[pl.ds(row0, CHUNK), :],
                sems_out[sl]).wait()

        start_in(0, 0)
        start_in(1, 1)

        def pair_body(p, carry):
            for sl in range(2):
                ck = 2 * p + sl
                wait_in(sl)

                @pl.when(p > 0)
                def _():
                    wait_out(sl)

                in_blk = in_v.at[sl]
                out_blk = out_v.at[sl]

                @plsc.parallel_loop(0, CHUNK, step=1, unroll=4)
                def row_body(r):
                    rvec = jnp.full((LANES,), r, jnp.int32)
                    for g in range(NGRP):
                        v = plsc.load_gather(in_blk, [rvec, idx_regs[g]])
                        out_blk[r, pl.ds(g * LANES, LANES)] = v

                start_out(ck, sl)

                @pl.when(p < NCHUNK // 2 - 1)
                def _():
                    start_in(ck + 2, sl)
            return carry

        lax.fori_loop(0, NCHUNK // 2, pair_body, 0)
        wait_out(0)
        wait_out(1)

    return body(values, indices)


def kernel(values, indices):
    return _sc_feature_select(values, indices)
